# baseline (speedup 1.0000x reference)
import jax
import jax.numpy as jnp
from jax import lax
from jax.experimental import pallas as pl
from jax.experimental.pallas import tpu as pltpu
from jax.experimental.pallas import tpu_sc as plsc

NUM_CORES = 2
NUM_SUBCORES = 16
NUM_WORKERS = NUM_CORES * NUM_SUBCORES

NUM_EMBEDDINGS = 1000000
BATCH = 16384
FIELDS = 26
EMBEDDING_DIM = 32
TOTAL = BATCH * FIELDS
LANES = 16


ACH = 256
FULL_CHUNKS = NUM_EMBEDDINGS // ACH
PER_WORKER_A = FULL_CHUNKS // NUM_WORKERS
EXTRA_CHUNKS = FULL_CHUNKS - PER_WORKER_A * NUM_WORKERS
TAIL_COLS = NUM_EMBEDDINGS - FULL_CHUNKS * ACH


def _relayout_body(tabT_hbm, tail_hbm, out_hbm, abuf0, abuf1, tbuf0, tbuf1,
                   mid, tailbuf, gsem0, gsem1, wsem0, wsem1):
  abufs = (abuf0, abuf1)
  tbufs = (tbuf0, tbuf1)
  gsems = (gsem0, gsem1)
  wsems = (wsem0, wsem1)

  wid = lax.axis_index("s") * NUM_CORES + lax.axis_index("c")
  iota = lax.iota(jnp.int32, LANES)
  zeros = jnp.zeros((LANES,), jnp.int32)
  vvecs = [iota + g * LANES for g in range(ACH // LANES)]

  def c0_of(k):
    return pl.multiple_of((wid + k * NUM_WORKERS) * ACH, ACH)

  def r0_of(k):
    return pl.multiple_of((wid + k * NUM_WORKERS) * (ACH // 4), ACH // 4)

  def start_in(k, b):
    return pltpu.async_copy(tabT_hbm.at[:, pl.ds(c0_of(k), ACH)],
                            abufs[b], gsems[b])

  def wait_in(k, b):
    pltpu.make_async_copy(tabT_hbm.at[:, pl.ds(c0_of(k), ACH)],
                          abufs[b], gsems[b]).wait()

  def transpose_chunk(b):
    ab, tb = abufs[b], tbufs[b]

    def dbody(d, carry):
      vs = [ab[d, pl.ds(g * LANES, LANES)] for g in range(ACH // LANES)]
      dfull = zeros + d
      for g in range(ACH // LANES):
        plsc.store_scatter(mid, [vvecs[g], dfull], vs[g])
      return carry

    lax.fori_loop(0, EMBEDDING_DIM, dbody, 0)

    def rbody(g, carry):
      vs = []
      for i in range(LANES):
        v = g * LANES + i
        vs.append(mid[v, pl.ds(0, LANES)])
        vs.append(mid[v, pl.ds(LANES, LANES)])
      for i in range(LANES):
        r = g * 4 + i // 4
        c = (i % 4) * EMBEDDING_DIM
        tb[r, pl.ds(c, LANES)] = vs[2 * i]
        tb[r, pl.ds(c + LANES, LANES)] = vs[2 * i + 1]
      return carry

    lax.fori_loop(0, ACH // LANES, rbody, 0)

  def start_out(k, b):
    return pltpu.async_copy(
        tbufs[b], out_hbm.at[pl.ds(r0_of(k), ACH // 4), :], wsems[b])

  def wait_out(k, b):
    pltpu.make_async_copy(
        tbufs[b], out_hbm.at[pl.ds(r0_of(k), ACH // 4), :], wsems[b]).wait()

  start_in(0, 0)
  start_in(1, 1)
  for k in (0, 1):
    b = k & 1
    wait_in(k, b)
    transpose_chunk(b)
    start_out(k, b)
    start_in(k + 2, b)

  def step(i2, carry):
    k = i2 * 2 + 2
    for b in (0, 1):
      kk = k + b
      wait_in(kk, b)
      wait_out(kk - 2, b)
      transpose_chunk(b)
      start_out(kk, b)
      start_in(kk + 2, b)
    return carry

  lax.fori_loop(0, (PER_WORKER_A - 4) // 2, step, 0)

  for k in (PER_WORKER_A - 2, PER_WORKER_A - 1):
    b = k & 1
    wait_in(k, b)
    wait_out(k - 2, b)
    transpose_chunk(b)
    start_out(k, b)
  for k in (PER_WORKER_A - 2, PER_WORKER_A - 1):
    wait_out(k, k & 1)

  for e in range(EXTRA_CHUNKS):
    c0 = (PER_WORKER_A * NUM_WORKERS + e) * ACH

    @pl.when(wid == e)
    def _():
      pltpu.sync_copy(tabT_hbm.at[:, pl.ds(c0, ACH)], abufs[0])
      transpose_chunk(0)
      pltpu.sync_copy(tbufs[0], out_hbm.at[pl.ds(c0 // 4, ACH // 4), :])

  if TAIL_COLS:
    c0t = FULL_CHUNKS * ACH

    @pl.when(wid == EXTRA_CHUNKS)
    def _():
      pltpu.sync_copy(tail_hbm, tailbuf)
      pltpu.sync_copy(tailbuf,
                      out_hbm.at[pl.ds(c0t // 4, TAIL_COLS // 4), :])


BBLK = 512
BLOCKS_PER_FIELD = BATCH // BBLK
NUM_TASKS = FIELDS * BLOCKS_PER_FIELD
PER_WORKER = NUM_TASKS // NUM_WORKERS
JSTEPS = BBLK // LANES


def _gather_body(table_hbm, idxT_hbm, out_hbm, idx_all, rows0, rows1,
                 trows0, trows1, gsem0, gsem1, wsem0, wsem1):
  rows = (rows0, rows1)
  trows = (trows0, trows1)
  gsems = (gsem0, gsem1)
  wsems = (wsem0, wsem1)

  wid = lax.axis_index("s") * NUM_CORES + lax.axis_index("c")
  task0 = wid * PER_WORKER
  pltpu.sync_copy(idxT_hbm.at[pl.ds(task0 * BBLK, PER_WORKER * BBLK)],
                  idx_all)

  iota = lax.iota(jnp.int32, LANES)
  dvecs = [jnp.full((LANES,), d, jnp.int32) for d in range(EMBEDDING_DIM)]

  zeros = jnp.zeros((LANES,), jnp.int32)
  iota_hi = iota + LANES

  def start_gather(t, b):
    return pltpu.async_copy(
        table_hbm.at[idx_all.at[pl.ds(t * BBLK, BBLK)]], rows[b], gsems[b])

  def wait_gather(t, b):
    pltpu.make_async_copy(
        table_hbm.at[idx_all.at[pl.ds(t * BBLK, BBLK)]], rows[b],
        gsems[b]).wait()

  def transpose_block(b):
    rb, tb = rows[b], trows[b]

    def gbody(g, carry):
      rbase = g * LANES
      vs = []
      for i in range(LANES):
        r = rbase + i
        vs.append(rb[r, pl.ds(0, LANES)])
        vs.append(rb[r, pl.ds(LANES, LANES)])
      for i in range(LANES):
        rfull = zeros + (rbase + i)
        plsc.store_scatter(tb, [iota, rfull], vs[2 * i])
        plsc.store_scatter(tb, [iota_hi, rfull], vs[2 * i + 1])
      return carry

    lax.fori_loop(0, JSTEPS, gbody, 0)

  def start_writeback(t, b):
    task = task0 + t
    f = task >> 5
    b0 = (task & 31) * BBLK
    return pltpu.async_copy(trows[b].at[:, pl.ds(0, BBLK)],
                            out_hbm.at[f, :, pl.ds(b0, BBLK)], wsems[b])

  def wait_writeback(t, b):
    task = task0 + t
    f = task >> 5
    b0 = (task & 31) * BBLK
    pltpu.make_async_copy(trows[b].at[:, pl.ds(0, BBLK)],
                          out_hbm.at[f, :, pl.ds(b0, BBLK)],
                          wsems[b]).wait()

  start_gather(0, 0)
  start_gather(1, 1)
  for t in (0, 1):
    b = t & 1
    wait_gather(t, b)
    transpose_block(b)
    start_writeback(t, b)
    start_gather(t + 2, b)

  def step(i2, carry):
    t = i2 * 2 + 2
    for b in (0, 1):
      tt = t + b
      wait_gather(tt, b)
      wait_writeback(tt - 2, b)
      transpose_block(b)
      start_writeback(tt, b)
      start_gather(tt + 2, b)
    return carry

  lax.fori_loop(0, (PER_WORKER - 4) // 2, step, 0)

  for t in (PER_WORKER - 2, PER_WORKER - 1):
    b = t & 1
    wait_gather(t, b)
    wait_writeback(t - 2, b)
    transpose_block(b)
    start_writeback(t, b)
  for t in (PER_WORKER - 2, PER_WORKER - 1):
    wait_writeback(t, t & 1)


@jax.jit
def kernel(x, table):
  mesh = plsc.VectorSubcoreMesh(core_axis_name="c", subcore_axis_name="s")

  tableT = table.T
  tail_lin = table[FULL_CHUNKS * ACH:].reshape(TAIL_COLS // 4,
                                               4 * EMBEDDING_DIM)
  table_rm4 = pl.kernel(
      _relayout_body,
      out_type=jax.ShapeDtypeStruct((NUM_EMBEDDINGS // 4,
                                     4 * EMBEDDING_DIM), jnp.float32),
      mesh=mesh,
      scratch_types=(
          [pltpu.VMEM((EMBEDDING_DIM, ACH), jnp.float32)] * 2
          + [pltpu.VMEM((ACH // 4, 4 * EMBEDDING_DIM), jnp.float32)] * 2
          + [pltpu.VMEM((ACH, EMBEDDING_DIM + 1), jnp.float32)]
          + [pltpu.VMEM((TAIL_COLS // 4, 4 * EMBEDDING_DIM), jnp.float32)]
          + [pltpu.SemaphoreType.DMA] * 4
      ),
      compiler_params=pltpu.CompilerParams(use_tc_tiling_on_sc=True,
                                           needs_layout_passes=False),
  )(tableT, tail_lin)
  table_rm = table_rm4.reshape(NUM_EMBEDDINGS, EMBEDDING_DIM)

  idxT = x.T.reshape(TOTAL)
  outP = pl.kernel(
      _gather_body,
      out_type=jax.ShapeDtypeStruct((FIELDS, EMBEDDING_DIM, BATCH),
                                    jnp.float32),
      mesh=mesh,
      scratch_types=(
          [pltpu.VMEM((PER_WORKER * BBLK,), jnp.int32)]
          + [pltpu.VMEM((BBLK, EMBEDDING_DIM), jnp.float32)] * 2
          + [pltpu.VMEM((EMBEDDING_DIM, BBLK + 1), jnp.float32)] * 2
          + [pltpu.SemaphoreType.DMA] * 4
      ),
      compiler_params=pltpu.CompilerParams(use_tc_tiling_on_sc=False,
                                           needs_layout_passes=False),
  )(table_rm, idxT)
  return jnp.transpose(outP, (2, 0, 1))


# speedup vs baseline: 2.1624x; 2.1624x over previous
import jax
import jax.numpy as jnp
from jax import lax
from jax.experimental import pallas as pl
from jax.experimental.pallas import tpu as pltpu
from jax.experimental.pallas import tpu_sc as plsc

NUM_CORES = 2
NUM_SUBCORES = 16
NUM_WORKERS = NUM_CORES * NUM_SUBCORES

NUM_EMBEDDINGS = 1000000
BATCH = 16384
FIELDS = 26
EMBEDDING_DIM = 32
TOTAL = BATCH * FIELDS
LANES = 16


ACH = 256
FULL_CHUNKS = NUM_EMBEDDINGS // ACH
PER_WORKER_A = FULL_CHUNKS // NUM_WORKERS
EXTRA_CHUNKS = FULL_CHUNKS - PER_WORKER_A * NUM_WORKERS
TAIL_COLS = NUM_EMBEDDINGS - FULL_CHUNKS * ACH


def _relayout_body(tabT_hbm, tail_hbm, out_hbm, abuf0, abuf1, tbuf0, tbuf1,
                   tailbuf, gsem0, gsem1, wsem0, wsem1):
  abufs = (abuf0, abuf1)
  tbufs = (tbuf0, tbuf1)
  gsems = (gsem0, gsem1)
  wsems = (wsem0, wsem1)

  wid = lax.axis_index("s") * NUM_CORES + lax.axis_index("c")
  iota = lax.iota(jnp.int32, LANES)
  zeros = jnp.zeros((LANES,), jnp.int32)
  iota_div4 = lax.shift_right_logical(iota, 2)
  colpart = lax.shift_left(iota & 3, 5)

  def c0_of(k):
    return pl.multiple_of((wid + k * NUM_WORKERS) * ACH, ACH)

  def r0_of(k):
    return pl.multiple_of((wid + k * NUM_WORKERS) * (ACH // 4), ACH // 4)

  def start_in(k, b):
    return pltpu.async_copy(tabT_hbm.at[:, pl.ds(c0_of(k), ACH)],
                            abufs[b], gsems[b])

  def wait_in(k, b):
    pltpu.make_async_copy(tabT_hbm.at[:, pl.ds(c0_of(k), ACH)],
                          abufs[b], gsems[b]).wait()

  def transpose_chunk(b):
    ab, tb = abufs[b], tbufs[b]

    def qbody(q, carry):
      gcol = iota + q * LANES
      srow = iota_div4 + q * 4
      for dbase in range(EMBEDDING_DIM):
        grow = (iota + dbase) & 31
        val = plsc.load_gather(ab, [grow, gcol])
        plsc.store_scatter(tb, [srow, colpart + grow], val)
      return carry

    lax.fori_loop(0, ACH // LANES, qbody, 0)

  def start_out(k, b):
    return pltpu.async_copy(
        tbufs[b], out_hbm.at[pl.ds(r0_of(k), ACH // 4), :], wsems[b])

  def wait_out(k, b):
    pltpu.make_async_copy(
        tbufs[b], out_hbm.at[pl.ds(r0_of(k), ACH // 4), :], wsems[b]).wait()

  start_in(0, 0)
  start_in(1, 1)
  for k in (0, 1):
    b = k & 1
    wait_in(k, b)
    transpose_chunk(b)
    start_out(k, b)
    start_in(k + 2, b)

  def step(i2, carry):
    k = i2 * 2 + 2
    for b in (0, 1):
      kk = k + b
      wait_in(kk, b)
      wait_out(kk - 2, b)
      transpose_chunk(b)
      start_out(kk, b)
      start_in(kk + 2, b)
    return carry

  lax.fori_loop(0, (PER_WORKER_A - 4) // 2, step, 0)

  for k in (PER_WORKER_A - 2, PER_WORKER_A - 1):
    b = k & 1
    wait_in(k, b)
    wait_out(k - 2, b)
    transpose_chunk(b)
    start_out(k, b)
  for k in (PER_WORKER_A - 2, PER_WORKER_A - 1):
    wait_out(k, k & 1)

  for e in range(EXTRA_CHUNKS):
    c0 = (PER_WORKER_A * NUM_WORKERS + e) * ACH

    @pl.when(wid == e)
    def _():
      pltpu.sync_copy(tabT_hbm.at[:, pl.ds(c0, ACH)], abufs[0])
      transpose_chunk(0)
      pltpu.sync_copy(tbufs[0], out_hbm.at[pl.ds(c0 // 4, ACH // 4), :])

  if TAIL_COLS:
    c0t = FULL_CHUNKS * ACH

    @pl.when(wid == EXTRA_CHUNKS)
    def _():
      pltpu.sync_copy(tail_hbm, tailbuf)
      pltpu.sync_copy(tailbuf,
                      out_hbm.at[pl.ds(c0t // 4, TAIL_COLS // 4), :])


BBLK = 512
BLOCKS_PER_FIELD = BATCH // BBLK
NUM_TASKS = FIELDS * BLOCKS_PER_FIELD
PER_WORKER = NUM_TASKS // NUM_WORKERS
JSTEPS = BBLK // LANES


def _gather_body(table_hbm, idxT_hbm, out_hbm, idx_all, rows0, rows1,
                 trows0, trows1, gsem0, gsem1, wsem0, wsem1):
  rows = (rows0, rows1)
  trows = (trows0, trows1)
  gsems = (gsem0, gsem1)
  wsems = (wsem0, wsem1)

  wid = lax.axis_index("s") * NUM_CORES + lax.axis_index("c")
  task0 = wid * PER_WORKER
  pltpu.sync_copy(idxT_hbm.at[pl.ds(task0 * BBLK, PER_WORKER * BBLK)],
                  idx_all)

  iota = lax.iota(jnp.int32, LANES)
  dvecs = [jnp.full((LANES,), d, jnp.int32) for d in range(EMBEDDING_DIM)]

  zeros = jnp.zeros((LANES,), jnp.int32)
  iota_hi = iota + LANES

  def start_gather(t, b):
    return pltpu.async_copy(
        table_hbm.at[idx_all.at[pl.ds(t * BBLK, BBLK)]], rows[b], gsems[b])

  def wait_gather(t, b):
    pltpu.make_async_copy(
        table_hbm.at[idx_all.at[pl.ds(t * BBLK, BBLK)]], rows[b],
        gsems[b]).wait()

  def transpose_block(b):
    rb, tb = rows[b], trows[b]

    def gbody(g, carry):
      rbase = g * LANES
      vs = []
      for i in range(LANES):
        r = rbase + i
        vs.append(rb[r, pl.ds(0, LANES)])
        vs.append(rb[r, pl.ds(LANES, LANES)])
      for i in range(LANES):
        rfull = zeros + (rbase + i)
        plsc.store_scatter(tb, [iota, rfull], vs[2 * i])
        plsc.store_scatter(tb, [iota_hi, rfull], vs[2 * i + 1])
      return carry

    lax.fori_loop(0, JSTEPS, gbody, 0)

  def start_writeback(t, b):
    task = task0 + t
    f = task >> 5
    b0 = (task & 31) * BBLK
    return pltpu.async_copy(trows[b].at[:, pl.ds(0, BBLK)],
                            out_hbm.at[f, :, pl.ds(b0, BBLK)], wsems[b])

  def wait_writeback(t, b):
    task = task0 + t
    f = task >> 5
    b0 = (task & 31) * BBLK
    pltpu.make_async_copy(trows[b].at[:, pl.ds(0, BBLK)],
                          out_hbm.at[f, :, pl.ds(b0, BBLK)],
                          wsems[b]).wait()

  start_gather(0, 0)
  start_gather(1, 1)
  for t in (0, 1):
    b = t & 1
    wait_gather(t, b)
    transpose_block(b)
    start_writeback(t, b)
    start_gather(t + 2, b)

  def step(i2, carry):
    t = i2 * 2 + 2
    for b in (0, 1):
      tt = t + b
      wait_gather(tt, b)
      wait_writeback(tt - 2, b)
      transpose_block(b)
      start_writeback(tt, b)
      start_gather(tt + 2, b)
    return carry

  lax.fori_loop(0, (PER_WORKER - 4) // 2, step, 0)

  for t in (PER_WORKER - 2, PER_WORKER - 1):
    b = t & 1
    wait_gather(t, b)
    wait_writeback(t - 2, b)
    transpose_block(b)
    start_writeback(t, b)
  for t in (PER_WORKER - 2, PER_WORKER - 1):
    wait_writeback(t, t & 1)


@jax.jit
def kernel(x, table):
  mesh = plsc.VectorSubcoreMesh(core_axis_name="c", subcore_axis_name="s")

  tableT = table.T
  tail_lin = table[FULL_CHUNKS * ACH:].reshape(TAIL_COLS // 4,
                                               4 * EMBEDDING_DIM)
  table_rm4 = pl.kernel(
      _relayout_body,
      out_type=jax.ShapeDtypeStruct((NUM_EMBEDDINGS // 4,
                                     4 * EMBEDDING_DIM), jnp.float32),
      mesh=mesh,
      scratch_types=(
          [pltpu.VMEM((EMBEDDING_DIM, ACH), jnp.float32)] * 2
          + [pltpu.VMEM((ACH // 4, 4 * EMBEDDING_DIM), jnp.float32)] * 2
          + [pltpu.VMEM((TAIL_COLS // 4, 4 * EMBEDDING_DIM), jnp.float32)]
          + [pltpu.SemaphoreType.DMA] * 4
      ),
      compiler_params=pltpu.CompilerParams(use_tc_tiling_on_sc=True,
                                           needs_layout_passes=False),
  )(tableT, tail_lin)
  table_rm = table_rm4.reshape(NUM_EMBEDDINGS, EMBEDDING_DIM)

  idxT = x.T.reshape(TOTAL)
  outP = pl.kernel(
      _gather_body,
      out_type=jax.ShapeDtypeStruct((FIELDS, EMBEDDING_DIM, BATCH),
                                    jnp.float32),
      mesh=mesh,
      scratch_types=(
          [pltpu.VMEM((PER_WORKER * BBLK,), jnp.int32)]
          + [pltpu.VMEM((BBLK, EMBEDDING_DIM), jnp.float32)] * 2
          + [pltpu.VMEM((EMBEDDING_DIM, BBLK + 1), jnp.float32)] * 2
          + [pltpu.SemaphoreType.DMA] * 4
      ),
      compiler_params=pltpu.CompilerParams(use_tc_tiling_on_sc=False,
                                           needs_layout_passes=False),
  )(table_rm, idxT)
  return jnp.transpose(outP, (2, 0, 1))


# speedup vs baseline: 2.1711x; 1.0040x over previous
import jax
import jax.numpy as jnp
from jax import lax
from jax.experimental import pallas as pl
from jax.experimental.pallas import tpu as pltpu
from jax.experimental.pallas import tpu_sc as plsc

NUM_CORES = 2
NUM_SUBCORES = 16
NUM_WORKERS = NUM_CORES * NUM_SUBCORES

NUM_EMBEDDINGS = 1000000
BATCH = 16384
FIELDS = 26
EMBEDDING_DIM = 32
TOTAL = BATCH * FIELDS
LANES = 16


ACH = 256
FULL_CHUNKS = NUM_EMBEDDINGS // ACH
PER_WORKER_A = FULL_CHUNKS // NUM_WORKERS
EXTRA_CHUNKS = FULL_CHUNKS - PER_WORKER_A * NUM_WORKERS
TAIL_COLS = NUM_EMBEDDINGS - FULL_CHUNKS * ACH


def _relayout_body(tabT_hbm, tail_hbm, out_hbm, abuf0, abuf1, tbuf0, tbuf1,
                   tailbuf, gsem0, gsem1, wsem0, wsem1):
  abufs = (abuf0, abuf1)
  tbufs = (tbuf0, tbuf1)
  gsems = (gsem0, gsem1)
  wsems = (wsem0, wsem1)

  wid = lax.axis_index("s") * NUM_CORES + lax.axis_index("c")
  iota = lax.iota(jnp.int32, LANES)
  zeros = jnp.zeros((LANES,), jnp.int32)
  iota_div4 = lax.shift_right_logical(iota, 2)
  colpart = lax.shift_left(iota & 3, 5)
  grows = [(iota + dbase) & 31 for dbase in range(EMBEDDING_DIM)]

  def c0_of(k):
    return pl.multiple_of((wid + k * NUM_WORKERS) * ACH, ACH)

  def r0_of(k):
    return pl.multiple_of((wid + k * NUM_WORKERS) * (ACH // 4), ACH // 4)

  def start_in(k, b):
    return pltpu.async_copy(tabT_hbm.at[:, pl.ds(c0_of(k), ACH)],
                            abufs[b], gsems[b])

  def wait_in(k, b):
    pltpu.make_async_copy(tabT_hbm.at[:, pl.ds(c0_of(k), ACH)],
                          abufs[b], gsems[b]).wait()

  def transpose_chunk(b):
    ab, tb = abufs[b], tbufs[b]

    def qbody(q, carry):
      gcol = iota + q * LANES
      srow = iota_div4 + q * 4
      for dbase in range(EMBEDDING_DIM):
        val = plsc.load_gather(ab, [grows[dbase], gcol])
        plsc.store_scatter(tb, [srow, colpart + grows[dbase]], val)
      return carry

    lax.fori_loop(0, ACH // LANES, qbody, 0)

  def start_out(k, b):
    return pltpu.async_copy(
        tbufs[b], out_hbm.at[pl.ds(r0_of(k), ACH // 4), :], wsems[b])

  def wait_out(k, b):
    pltpu.make_async_copy(
        tbufs[b], out_hbm.at[pl.ds(r0_of(k), ACH // 4), :], wsems[b]).wait()

  start_in(0, 0)
  start_in(1, 1)
  for k in (0, 1):
    b = k & 1
    wait_in(k, b)
    transpose_chunk(b)
    start_out(k, b)
    start_in(k + 2, b)

  def step(i2, carry):
    k = i2 * 2 + 2
    for b in (0, 1):
      kk = k + b
      wait_in(kk, b)
      wait_out(kk - 2, b)
      transpose_chunk(b)
      start_out(kk, b)
      start_in(kk + 2, b)
    return carry

  lax.fori_loop(0, (PER_WORKER_A - 4) // 2, step, 0)

  for k in (PER_WORKER_A - 2, PER_WORKER_A - 1):
    b = k & 1
    wait_in(k, b)
    wait_out(k - 2, b)
    transpose_chunk(b)
    start_out(k, b)
  for k in (PER_WORKER_A - 2, PER_WORKER_A - 1):
    wait_out(k, k & 1)

  for e in range(EXTRA_CHUNKS):
    c0 = (PER_WORKER_A * NUM_WORKERS + e) * ACH

    @pl.when(wid == e)
    def _():
      pltpu.sync_copy(tabT_hbm.at[:, pl.ds(c0, ACH)], abufs[0])
      transpose_chunk(0)
      pltpu.sync_copy(tbufs[0], out_hbm.at[pl.ds(c0 // 4, ACH // 4), :])

  if TAIL_COLS:
    c0t = FULL_CHUNKS * ACH

    @pl.when(wid == EXTRA_CHUNKS)
    def _():
      pltpu.sync_copy(tail_hbm, tailbuf)
      pltpu.sync_copy(tailbuf,
                      out_hbm.at[pl.ds(c0t // 4, TAIL_COLS // 4), :])


BBLK = 512
BLOCKS_PER_FIELD = BATCH // BBLK
NUM_TASKS = FIELDS * BLOCKS_PER_FIELD
PER_WORKER = NUM_TASKS // NUM_WORKERS
JSTEPS = BBLK // LANES


def _gather_body(table_hbm, idxT_hbm, out_hbm, idx_all, rows0, rows1,
                 trows0, trows1, gsem0, gsem1, wsem0, wsem1):
  rows = (rows0, rows1)
  trows = (trows0, trows1)
  gsems = (gsem0, gsem1)
  wsems = (wsem0, wsem1)

  wid = lax.axis_index("s") * NUM_CORES + lax.axis_index("c")
  task0 = wid * PER_WORKER
  pltpu.sync_copy(idxT_hbm.at[pl.ds(task0 * BBLK, PER_WORKER * BBLK)],
                  idx_all)

  iota = lax.iota(jnp.int32, LANES)
  dvecs = [jnp.full((LANES,), d, jnp.int32) for d in range(EMBEDDING_DIM)]

  zeros = jnp.zeros((LANES,), jnp.int32)
  iota_hi = iota + LANES

  def start_gather(t, b):
    return pltpu.async_copy(
        table_hbm.at[idx_all.at[pl.ds(t * BBLK, BBLK)]], rows[b], gsems[b])

  def wait_gather(t, b):
    pltpu.make_async_copy(
        table_hbm.at[idx_all.at[pl.ds(t * BBLK, BBLK)]], rows[b],
        gsems[b]).wait()

  def transpose_block(b):
    rb, tb = rows[b], trows[b]

    def gbody(g, carry):
      rbase = g * LANES
      vs = []
      for i in range(LANES):
        r = rbase + i
        vs.append(rb[r, pl.ds(0, LANES)])
        vs.append(rb[r, pl.ds(LANES, LANES)])
      for i in range(LANES):
        rfull = zeros + (rbase + i)
        plsc.store_scatter(tb, [iota, rfull], vs[2 * i])
        plsc.store_scatter(tb, [iota_hi, rfull], vs[2 * i + 1])
      return carry

    lax.fori_loop(0, JSTEPS, gbody, 0)

  def start_writeback(t, b):
    task = task0 + t
    f = task >> 5
    b0 = (task & 31) * BBLK
    return pltpu.async_copy(trows[b].at[:, pl.ds(0, BBLK)],
                            out_hbm.at[f, :, pl.ds(b0, BBLK)], wsems[b])

  def wait_writeback(t, b):
    task = task0 + t
    f = task >> 5
    b0 = (task & 31) * BBLK
    pltpu.make_async_copy(trows[b].at[:, pl.ds(0, BBLK)],
                          out_hbm.at[f, :, pl.ds(b0, BBLK)],
                          wsems[b]).wait()

  start_gather(0, 0)
  start_gather(1, 1)
  for t in (0, 1):
    b = t & 1
    wait_gather(t, b)
    transpose_block(b)
    start_writeback(t, b)
    start_gather(t + 2, b)

  def step(i2, carry):
    t = i2 * 2 + 2
    for b in (0, 1):
      tt = t + b
      wait_gather(tt, b)
      wait_writeback(tt - 2, b)
      transpose_block(b)
      start_writeback(tt, b)
      start_gather(tt + 2, b)
    return carry

  lax.fori_loop(0, (PER_WORKER - 4) // 2, step, 0)

  for t in (PER_WORKER - 2, PER_WORKER - 1):
    b = t & 1
    wait_gather(t, b)
    wait_writeback(t - 2, b)
    transpose_block(b)
    start_writeback(t, b)
  for t in (PER_WORKER - 2, PER_WORKER - 1):
    wait_writeback(t, t & 1)


@jax.jit
def kernel(x, table):
  mesh = plsc.VectorSubcoreMesh(core_axis_name="c", subcore_axis_name="s")

  tableT = table.T
  tail_lin = table[FULL_CHUNKS * ACH:].reshape(TAIL_COLS // 4,
                                               4 * EMBEDDING_DIM)
  table_rm4 = pl.kernel(
      _relayout_body,
      out_type=jax.ShapeDtypeStruct((NUM_EMBEDDINGS // 4,
                                     4 * EMBEDDING_DIM), jnp.float32),
      mesh=mesh,
      scratch_types=(
          [pltpu.VMEM((EMBEDDING_DIM, ACH), jnp.float32)] * 2
          + [pltpu.VMEM((ACH // 4, 4 * EMBEDDING_DIM), jnp.float32)] * 2
          + [pltpu.VMEM((TAIL_COLS // 4, 4 * EMBEDDING_DIM), jnp.float32)]
          + [pltpu.SemaphoreType.DMA] * 4
      ),
      compiler_params=pltpu.CompilerParams(use_tc_tiling_on_sc=True,
                                           needs_layout_passes=False),
  )(tableT, tail_lin)
  table_rm = table_rm4.reshape(NUM_EMBEDDINGS, EMBEDDING_DIM)

  idxT = x.T.reshape(TOTAL)
  outP = pl.kernel(
      _gather_body,
      out_type=jax.ShapeDtypeStruct((FIELDS, EMBEDDING_DIM, BATCH),
                                    jnp.float32),
      mesh=mesh,
      scratch_types=(
          [pltpu.VMEM((PER_WORKER * BBLK,), jnp.int32)]
          + [pltpu.VMEM((BBLK, EMBEDDING_DIM), jnp.float32)] * 2
          + [pltpu.VMEM((EMBEDDING_DIM, BBLK + 1), jnp.float32)] * 2
          + [pltpu.SemaphoreType.DMA] * 4
      ),
      compiler_params=pltpu.CompilerParams(use_tc_tiling_on_sc=False,
                                           needs_layout_passes=False),
  )(table_rm, idxT)
  return jnp.transpose(outP, (2, 0, 1))


# speedup vs baseline: 3.1836x; 1.4664x over previous
import jax
import jax.numpy as jnp
from jax import lax
from jax.experimental import pallas as pl
from jax.experimental.pallas import tpu as pltpu
from jax.experimental.pallas import tpu_sc as plsc

NUM_CORES = 2
NUM_SUBCORES = 16
NUM_WORKERS = NUM_CORES * NUM_SUBCORES

NUM_EMBEDDINGS = 1000000
BATCH = 16384
FIELDS = 26
EMBEDDING_DIM = 32
TOTAL = BATCH * FIELDS
LANES = 16


ACH = 256
FULL_CHUNKS = NUM_EMBEDDINGS // ACH
PER_WORKER_A = FULL_CHUNKS // NUM_WORKERS
EXTRA_CHUNKS = FULL_CHUNKS - PER_WORKER_A * NUM_WORKERS
TAIL_COLS = NUM_EMBEDDINGS - FULL_CHUNKS * ACH


def _relayout_body(tabT_hbm, tail_hbm, out_hbm, abuf0, abuf1, tbuf0, tbuf1,
                   tailbuf, gsem0, gsem1, wsem0, wsem1):
  abufs = (abuf0, abuf1)
  tbufs = (tbuf0, tbuf1)
  gsems = (gsem0, gsem1)
  wsems = (wsem0, wsem1)

  wid = lax.axis_index("s") * NUM_CORES + lax.axis_index("c")
  iota = lax.iota(jnp.int32, LANES)
  zeros = jnp.zeros((LANES,), jnp.int32)
  iota_div4 = lax.shift_right_logical(iota, 2)
  colpart = lax.shift_left(iota & 3, 5)
  grows = [(iota + dbase) & 31 for dbase in range(EMBEDDING_DIM)]

  def c0_of(k):
    return pl.multiple_of((wid + k * NUM_WORKERS) * ACH, ACH)

  def r0_of(k):
    return pl.multiple_of((wid + k * NUM_WORKERS) * (ACH // 4), ACH // 4)

  def start_in(k, b):
    return pltpu.async_copy(tabT_hbm.at[:, pl.ds(c0_of(k), ACH)],
                            abufs[b], gsems[b])

  def wait_in(k, b):
    pltpu.make_async_copy(tabT_hbm.at[:, pl.ds(c0_of(k), ACH)],
                          abufs[b], gsems[b]).wait()

  def transpose_chunk(b):
    ab, tb = abufs[b], tbufs[b]

    def qbody(q, carry):
      gcol = iota + q * LANES
      srow = iota_div4 + q * 4
      for db0 in range(0, EMBEDDING_DIM, 8):
        vals = [plsc.load_gather(ab, [grows[db0 + j], gcol])
                for j in range(8)]
        for j in range(8):
          plsc.store_scatter(tb, [srow, colpart + grows[db0 + j]], vals[j])
      return carry

    lax.fori_loop(0, ACH // LANES, qbody, 0)

  def start_out(k, b):
    return pltpu.async_copy(
        tbufs[b], out_hbm.at[pl.ds(r0_of(k), ACH // 4), :], wsems[b])

  def wait_out(k, b):
    pltpu.make_async_copy(
        tbufs[b], out_hbm.at[pl.ds(r0_of(k), ACH // 4), :], wsems[b]).wait()

  start_in(0, 0)
  start_in(1, 1)
  for k in (0, 1):
    b = k & 1
    wait_in(k, b)
    transpose_chunk(b)
    start_out(k, b)
    start_in(k + 2, b)

  def step(i2, carry):
    k = i2 * 2 + 2
    for b in (0, 1):
      kk = k + b
      wait_in(kk, b)
      wait_out(kk - 2, b)
      transpose_chunk(b)
      start_out(kk, b)
      start_in(kk + 2, b)
    return carry

  lax.fori_loop(0, (PER_WORKER_A - 4) // 2, step, 0)

  for k in (PER_WORKER_A - 2, PER_WORKER_A - 1):
    b = k & 1
    wait_in(k, b)
    wait_out(k - 2, b)
    transpose_chunk(b)
    start_out(k, b)
  for k in (PER_WORKER_A - 2, PER_WORKER_A - 1):
    wait_out(k, k & 1)

  for e in range(EXTRA_CHUNKS):
    c0 = (PER_WORKER_A * NUM_WORKERS + e) * ACH

    @pl.when(wid == e)
    def _():
      pltpu.sync_copy(tabT_hbm.at[:, pl.ds(c0, ACH)], abufs[0])
      transpose_chunk(0)
      pltpu.sync_copy(tbufs[0], out_hbm.at[pl.ds(c0 // 4, ACH // 4), :])

  if TAIL_COLS:
    c0t = FULL_CHUNKS * ACH

    @pl.when(wid == EXTRA_CHUNKS)
    def _():
      pltpu.sync_copy(tail_hbm, tailbuf)
      pltpu.sync_copy(tailbuf,
                      out_hbm.at[pl.ds(c0t // 4, TAIL_COLS // 4), :])


BBLK = 512
BLOCKS_PER_FIELD = BATCH // BBLK
NUM_TASKS = FIELDS * BLOCKS_PER_FIELD
PER_WORKER = NUM_TASKS // NUM_WORKERS
JSTEPS = BBLK // LANES


def _gather_body(table_hbm, idxT_hbm, out_hbm, idx_all, rows0, rows1,
                 trows0, trows1, gsem0, gsem1, wsem0, wsem1):
  rows = (rows0, rows1)
  trows = (trows0, trows1)
  gsems = (gsem0, gsem1)
  wsems = (wsem0, wsem1)

  wid = lax.axis_index("s") * NUM_CORES + lax.axis_index("c")
  task0 = wid * PER_WORKER
  pltpu.sync_copy(idxT_hbm.at[pl.ds(task0 * BBLK, PER_WORKER * BBLK)],
                  idx_all)

  iota = lax.iota(jnp.int32, LANES)
  dvecs = [jnp.full((LANES,), d, jnp.int32) for d in range(EMBEDDING_DIM)]

  zeros = jnp.zeros((LANES,), jnp.int32)
  iota_hi = iota + LANES

  def start_gather(t, b):
    return pltpu.async_copy(
        table_hbm.at[idx_all.at[pl.ds(t * BBLK, BBLK)]], rows[b], gsems[b])

  def wait_gather(t, b):
    pltpu.make_async_copy(
        table_hbm.at[idx_all.at[pl.ds(t * BBLK, BBLK)]], rows[b],
        gsems[b]).wait()

  def transpose_block(b):
    rb, tb = rows[b], trows[b]

    def gbody(g, carry):
      rbase = g * LANES
      vs = []
      for i in range(LANES):
        r = rbase + i
        vs.append(rb[r, pl.ds(0, LANES)])
        vs.append(rb[r, pl.ds(LANES, LANES)])
      for i in range(LANES):
        rfull = zeros + (rbase + i)
        plsc.store_scatter(tb, [iota, rfull], vs[2 * i])
        plsc.store_scatter(tb, [iota_hi, rfull], vs[2 * i + 1])
      return carry

    lax.fori_loop(0, JSTEPS, gbody, 0)

  def start_writeback(t, b):
    task = task0 + t
    f = task >> 5
    b0 = (task & 31) * BBLK
    return pltpu.async_copy(trows[b].at[:, pl.ds(0, BBLK)],
                            out_hbm.at[f, :, pl.ds(b0, BBLK)], wsems[b])

  def wait_writeback(t, b):
    task = task0 + t
    f = task >> 5
    b0 = (task & 31) * BBLK
    pltpu.make_async_copy(trows[b].at[:, pl.ds(0, BBLK)],
                          out_hbm.at[f, :, pl.ds(b0, BBLK)],
                          wsems[b]).wait()

  start_gather(0, 0)
  start_gather(1, 1)
  for t in (0, 1):
    b = t & 1
    wait_gather(t, b)
    transpose_block(b)
    start_writeback(t, b)
    start_gather(t + 2, b)

  def step(i2, carry):
    t = i2 * 2 + 2
    for b in (0, 1):
      tt = t + b
      wait_gather(tt, b)
      wait_writeback(tt - 2, b)
      transpose_block(b)
      start_writeback(tt, b)
      start_gather(tt + 2, b)
    return carry

  lax.fori_loop(0, (PER_WORKER - 4) // 2, step, 0)

  for t in (PER_WORKER - 2, PER_WORKER - 1):
    b = t & 1
    wait_gather(t, b)
    wait_writeback(t - 2, b)
    transpose_block(b)
    start_writeback(t, b)
  for t in (PER_WORKER - 2, PER_WORKER - 1):
    wait_writeback(t, t & 1)


@jax.jit
def kernel(x, table):
  mesh = plsc.VectorSubcoreMesh(core_axis_name="c", subcore_axis_name="s")

  tableT = table.T
  tail_lin = table[FULL_CHUNKS * ACH:].reshape(TAIL_COLS // 4,
                                               4 * EMBEDDING_DIM)
  table_rm4 = pl.kernel(
      _relayout_body,
      out_type=jax.ShapeDtypeStruct((NUM_EMBEDDINGS // 4,
                                     4 * EMBEDDING_DIM), jnp.float32),
      mesh=mesh,
      scratch_types=(
          [pltpu.VMEM((EMBEDDING_DIM, ACH), jnp.float32)] * 2
          + [pltpu.VMEM((ACH // 4, 4 * EMBEDDING_DIM), jnp.float32)] * 2
          + [pltpu.VMEM((TAIL_COLS // 4, 4 * EMBEDDING_DIM), jnp.float32)]
          + [pltpu.SemaphoreType.DMA] * 4
      ),
      compiler_params=pltpu.CompilerParams(use_tc_tiling_on_sc=True,
                                           needs_layout_passes=False),
  )(tableT, tail_lin)
  table_rm = table_rm4.reshape(NUM_EMBEDDINGS, EMBEDDING_DIM)

  idxT = x.T.reshape(TOTAL)
  outP = pl.kernel(
      _gather_body,
      out_type=jax.ShapeDtypeStruct((FIELDS, EMBEDDING_DIM, BATCH),
                                    jnp.float32),
      mesh=mesh,
      scratch_types=(
          [pltpu.VMEM((PER_WORKER * BBLK,), jnp.int32)]
          + [pltpu.VMEM((BBLK, EMBEDDING_DIM), jnp.float32)] * 2
          + [pltpu.VMEM((EMBEDDING_DIM, BBLK + 1), jnp.float32)] * 2
          + [pltpu.SemaphoreType.DMA] * 4
      ),
      compiler_params=pltpu.CompilerParams(use_tc_tiling_on_sc=False,
                                           needs_layout_passes=False),
  )(table_rm, idxT)
  return jnp.transpose(outP, (2, 0, 1))
